# baseline (speedup 1.0000x reference)
import jax
import jax.numpy as jnp
from jax import lax
from jax.experimental import pallas as pl
from jax.experimental.pallas import tpu as pltpu
from jax.experimental.pallas import tpu_sc as plsc

VOCAB = 1000000
HIDDEN = 32
EMB = 128
BATCH = 16384
HIST = 50
HISTP = 56
NTOK = BATCH * HIST
NTOKP = BATCH * HISTP


_INFO = plsc.get_sparse_core_info()
_NC = _INFO.num_cores
_NS = _INFO.num_subcores
_NW = _NC * _NS
_BATCH_PER_W = BATCH // _NW
_CB = 16
_CTOK = _CB * HIST
_NCHUNK = _BATCH_PER_W // _CB
_GATHERS = ((0, 128), (128, 128), (256, 128), (384, 128),
            (512, 128), (640, 128), (768, 32))


def _sc_gather_body(idx_hbm, table_hbm, h_hbm, idx_v, rows_v, sem):
    wid = lax.axis_index("s") * _NC + lax.axis_index("c")
    b0 = wid * _BATCH_PER_W

    def chunk(c, carry):
        bb = b0 + c * _CB
        pltpu.sync_copy(idx_hbm.at[pl.ds(bb * HIST, _CTOK)], idx_v)
        copies = []
        for off, n in _GATHERS:
            copies.append(pltpu.async_copy(
                table_hbm.at[idx_v.at[pl.ds(off, n)]],
                rows_v.at[pl.ds(off, n)],
                sem,
            ))
        for cp in copies:
            cp.wait()
        for b in range(_CB):
            pltpu.sync_copy(
                rows_v.at[pl.ds(b * HIST, HIST)],
                h_hbm.at[pl.ds((bb + b) * HISTP, HIST)],
            )
        return carry

    lax.fori_loop(0, _NCHUNK, chunk, 0)


def _sc_gather(x_flat, u_weight):
    mesh = plsc.VectorSubcoreMesh(core_axis_name="c", subcore_axis_name="s")
    k = pl.kernel(
        _sc_gather_body,
        out_type=jax.ShapeDtypeStruct((NTOKP, HIDDEN), jnp.float32),
        mesh=mesh,
        scratch_types=[
            pltpu.VMEM((_CTOK,), jnp.int32),
            pltpu.VMEM((_CTOK, HIDDEN), jnp.float32),
            pltpu.SemaphoreType.DMA,
        ],
        compiler_params=pltpu.CompilerParams(use_tc_tiling_on_sc=False),
    )
    return k(x_flat, u_weight)


_MROW = 1792


def _mm_body(hp_ref, v4_ref, b4_ref, o_ref):
    o_ref[...] = (
        jnp.dot(hp_ref[...], v4_ref[...], preferred_element_type=jnp.float32)
        + b4_ref[...]
    )


def _tc_project(hp, v_weight, v_bias):
    v4 = jnp.kron(jnp.eye(4, dtype=jnp.float32), v_weight)
    b4 = jnp.tile(v_bias, 4).reshape(1, 4 * EMB)
    rows = NTOKP // 4
    return pl.pallas_call(
        _mm_body,
        grid=(rows // _MROW,),
        in_specs=[
            pl.BlockSpec((_MROW, EMB), lambda i: (i, 0)),
            pl.BlockSpec((EMB, 4 * EMB), lambda i: (0, 0)),
            pl.BlockSpec((1, 4 * EMB), lambda i: (0, 0)),
        ],
        out_specs=pl.BlockSpec((_MROW, 4 * EMB), lambda i: (i, 0)),
        out_shape=jax.ShapeDtypeStruct((rows, 4 * EMB), jnp.float32),
    )(hp, v4, b4)


@jax.jit
def kernel(x, u_weight, v_weight, v_bias):
    x_flat = x.reshape(-1).astype(jnp.int32)
    h56 = _sc_gather(x_flat, u_weight)
    hp = h56.reshape(NTOKP // 4, 4 * HIDDEN)
    yp = _tc_project(hp, v_weight, v_bias)
    return yp.reshape(BATCH, HISTP, EMB)[:, :HIST, :]


# speedup vs baseline: 2.8872x; 2.8872x over previous
import jax
import jax.numpy as jnp
from jax import lax
from jax.experimental import pallas as pl
from jax.experimental.pallas import tpu as pltpu
from jax.experimental.pallas import tpu_sc as plsc

VOCAB = 1000000
HIDDEN = 32
EMB = 128
BATCH = 16384
HIST = 50
NTOK = BATCH * HIST


_WBLK = 8192


def _wb_body(ut_ref, v_ref, b_ref, w_ref):
    w_ref[...] = (
        lax.dot_general(
            ut_ref[...], v_ref[...],
            (((0,), (0,)), ((), ())),
            preferred_element_type=jnp.float32,
        )
        + b_ref[...]
    )


def _build_w(ut, v, b):
    return pl.pallas_call(
        _wb_body,
        grid=((VOCAB + _WBLK - 1) // _WBLK,),
        in_specs=[
            pl.BlockSpec((HIDDEN, _WBLK), lambda i: (0, i)),
            pl.BlockSpec((HIDDEN, EMB), lambda i: (0, 0)),
            pl.BlockSpec((1, EMB), lambda i: (0, 0)),
        ],
        out_specs=pl.BlockSpec((_WBLK, EMB), lambda i: (i, 0)),
        out_shape=jax.ShapeDtypeStruct((VOCAB, EMB), jnp.float32),
    )(ut, v, b.reshape(1, EMB))


_INFO = plsc.get_sparse_core_info()
_NC = _INFO.num_cores
_NS = _INFO.num_subcores
_NW = _NC * _NS
_TOK_PER_W = NTOK // _NW
_CHUNK = 800
_NCHUNK = _TOK_PER_W // _CHUNK
_GATHERS = ((0, 128), (128, 128), (256, 128), (384, 128),
            (512, 128), (640, 128), (768, 32))


def _sc_body(idx_hbm, w_hbm, out_hbm, idx_v, rows_v, sem):
    wid = lax.axis_index("s") * _NC + lax.axis_index("c")
    base = wid * _TOK_PER_W

    def chunk(c, carry):
        off = base + c * _CHUNK
        pltpu.sync_copy(idx_hbm.at[pl.ds(off, _CHUNK)], idx_v)
        copies = []
        for o, n in _GATHERS:
            copies.append(pltpu.async_copy(
                w_hbm.at[idx_v.at[pl.ds(o, n)]],
                rows_v.at[pl.ds(o, n)],
                sem,
            ))
        for cp in copies:
            cp.wait()
        pltpu.sync_copy(rows_v, out_hbm.at[pl.ds(off, _CHUNK)])
        return carry

    lax.fori_loop(0, _NCHUNK, chunk, 0)


def _sc_gather(idx_flat, w):
    mesh = plsc.VectorSubcoreMesh(core_axis_name="c", subcore_axis_name="s")
    k = pl.kernel(
        _sc_body,
        out_type=jax.ShapeDtypeStruct((NTOK, EMB), jnp.float32),
        mesh=mesh,
        scratch_types=[
            pltpu.VMEM((_CHUNK,), jnp.int32),
            pltpu.VMEM((_CHUNK, EMB), jnp.float32),
            pltpu.SemaphoreType.DMA,
        ],
        compiler_params=pltpu.CompilerParams(use_tc_tiling_on_sc=True),
    )
    return k(idx_flat, w)


@jax.jit
def kernel(x, u_weight, v_weight, v_bias):
    idx_flat = x.T.reshape(-1).astype(jnp.int32)
    w = _build_w(u_weight.T, v_weight, v_bias)
    out2d = _sc_gather(idx_flat, w)
    return out2d.reshape(HIST, BATCH, EMB).transpose(1, 0, 2)


# speedup vs baseline: 3.1679x; 1.0972x over previous
import jax
import jax.numpy as jnp
from jax import lax
from jax.experimental import pallas as pl
from jax.experimental.pallas import tpu as pltpu
from jax.experimental.pallas import tpu_sc as plsc

VOCAB = 1000000
HIDDEN = 32
EMB = 128
BATCH = 16384
HIST = 50
NTOK = BATCH * HIST


_WBLK = 16384


def _wb_body(ut_ref, v_ref, b_ref, w_ref):
    w_ref[...] = (
        lax.dot_general(
            ut_ref[...], v_ref[...],
            (((0,), (0,)), ((), ())),
            preferred_element_type=jnp.float32,
        )
        + b_ref[...]
    )


def _build_w(ut, v, b):
    return pl.pallas_call(
        _wb_body,
        grid=((VOCAB + _WBLK - 1) // _WBLK,),
        in_specs=[
            pl.BlockSpec((HIDDEN, _WBLK), lambda i: (0, i)),
            pl.BlockSpec((HIDDEN, EMB), lambda i: (0, 0)),
            pl.BlockSpec((1, EMB), lambda i: (0, 0)),
        ],
        out_specs=pl.BlockSpec((_WBLK, EMB), lambda i: (i, 0)),
        out_shape=jax.ShapeDtypeStruct((VOCAB, EMB), jnp.float32),
    )(ut, v, b.reshape(1, EMB))


_INFO = plsc.get_sparse_core_info()
_NC = _INFO.num_cores
_NS = _INFO.num_subcores
_NW = _NC * _NS
_TOK_PER_W = NTOK // _NW
_CHUNK = 400
_NCHUNK = _TOK_PER_W // _CHUNK
_GATHERS = ((0, 128), (128, 128), (256, 128), (384, 16))


def _sc_body(idx_hbm, w_hbm, out_hbm,
             idx0, idx1, rows0, rows1, sg0, sg1, sw0, sw1):
    wid = lax.axis_index("s") * _NC + lax.axis_index("c")
    base = wid * _TOK_PER_W

    def g_descs(idxv, rowsv, sem):
        return [
            pltpu.make_async_copy(
                w_hbm.at[idxv.at[pl.ds(o, n)]],
                rowsv.at[pl.ds(o, n)],
                sem,
            )
            for o, n in _GATHERS
        ]

    def w_desc(c, rowsv, sem):
        return pltpu.make_async_copy(
            rowsv, out_hbm.at[pl.ds(base + c * _CHUNK, _CHUNK)], sem)

    def load_idx(c, idxv):
        pltpu.sync_copy(idx_hbm.at[pl.ds(base + c * _CHUNK, _CHUNK)], idxv)

    def pair(p, carry):
        c0 = 2 * p
        load_idx(c0, idx0)

        @pl.when(p >= 1)
        def _():
            w_desc(c0 - 2, rows0, sw0).wait()

        for d in g_descs(idx0, rows0, sg0):
            d.start()

        @pl.when(p >= 1)
        def _():
            for d in g_descs(idx1, rows1, sg1):
                d.wait()
            w_desc(c0 - 1, rows1, sw1).start()

        load_idx(c0 + 1, idx1)

        @pl.when(p >= 1)
        def _():
            w_desc(c0 - 1, rows1, sw1).wait()

        for d in g_descs(idx1, rows1, sg1):
            d.start()

        for d in g_descs(idx0, rows0, sg0):
            d.wait()
        w_desc(c0, rows0, sw0).start()
        return carry

    lax.fori_loop(0, _NCHUNK // 2, pair, 0)

    for d in g_descs(idx1, rows1, sg1):
        d.wait()
    w_desc(_NCHUNK - 1, rows1, sw1).start()
    w_desc(_NCHUNK - 2, rows0, sw0).wait()
    w_desc(_NCHUNK - 1, rows1, sw1).wait()


def _sc_gather(idx_flat, w):
    mesh = plsc.VectorSubcoreMesh(core_axis_name="c", subcore_axis_name="s")
    k = pl.kernel(
        _sc_body,
        out_type=jax.ShapeDtypeStruct((NTOK, EMB), jnp.float32),
        mesh=mesh,
        scratch_types=[
            pltpu.VMEM((_CHUNK,), jnp.int32),
            pltpu.VMEM((_CHUNK,), jnp.int32),
            pltpu.VMEM((_CHUNK, EMB), jnp.float32),
            pltpu.VMEM((_CHUNK, EMB), jnp.float32),
            pltpu.SemaphoreType.DMA,
            pltpu.SemaphoreType.DMA,
            pltpu.SemaphoreType.DMA,
            pltpu.SemaphoreType.DMA,
        ],
        compiler_params=pltpu.CompilerParams(use_tc_tiling_on_sc=True),
    )
    return k(idx_flat, w)


@jax.jit
def kernel(x, u_weight, v_weight, v_bias):
    idx_flat = x.T.reshape(-1).astype(jnp.int32)
    w = _build_w(u_weight.T, v_weight, v_bias)
    out2d = _sc_gather(idx_flat, w)
    return out2d.reshape(HIST, BATCH, EMB).transpose(1, 0, 2)
